# Initial kernel scaffold; baseline (speedup 1.0000x reference)
import jax
import jax.numpy as jnp
from jax.experimental import pallas as pl


def kernel(nodes, edges, globals_, senders, receivers, n_node, n_edge, edge_weights, W_node, b_node, W_edge, b_edge, W_gn, b_gn, W_ge, b_ge, W_g, b_g, W_fg, b_fg):
    raise NotImplementedError("write your pallas kernel here")


# speedup vs baseline: 1.5793x; 1.5793x over previous
import functools

import jax
import jax.numpy as jnp
from jax import lax
from jax.experimental import pallas as pl
from jax.experimental.pallas import tpu as pltpu
from jax.experimental.pallas import tpu_sc as plsc

_NC = 2
_NS = 16
_PW = 160


def _pre_body(x_ref, g_ref, wcat_ref, w4_ref, p1_ref, p2_ref, q1_ref, q2_ref,
              gv_ref):
    y = jnp.dot(x_ref[...], wcat_ref[...], preferred_element_type=jnp.float32)
    d = p1_ref.shape[1]
    de = q1_ref.shape[1]
    p1_ref[...] = y[:, :d]
    p2_ref[...] = y[:, d:2 * d]
    q1_ref[...] = y[:, 2 * d:2 * d + de]
    q2_ref[...] = y[:, 2 * d + de:]

    @pl.when(pl.program_id(0) == 0)
    def _():
        gv_ref[...] = jnp.dot(g_ref[...], w4_ref[...],
                              preferred_element_type=jnp.float32)


def _make_sc_edge_pass(n, d, e, de):
    nw = _NC * _NS
    epw = e // nw
    ch = 40
    nit = epw // ch
    nch = n // ch
    npt = -(-nch // _NS)

    mesh = plsc.VectorSubcoreMesh(core_axis_name="c", subcore_axis_name="s",
                                  num_cores=_NC, num_subcores=_NS)

    @functools.partial(
        pl.kernel,
        out_type=(
            jax.ShapeDtypeStruct((_NC * n, _PW), jnp.float32),
            jax.ShapeDtypeStruct((e, de), jnp.float32),
        ),
        mesh=mesh,
        compiler_params=pltpu.CompilerParams(use_tc_tiling_on_sc=False),
        scratch_types=[
            pltpu.VMEM_SHARED((n, _PW), jnp.float32),
            pltpu.VMEM((ch,), jnp.int32),
            pltpu.VMEM((ch,), jnp.int32),
            pltpu.VMEM((ch,), jnp.float32),
            pltpu.VMEM((ch, de), jnp.float32),
            pltpu.VMEM((ch, d), jnp.float32),
            pltpu.VMEM((ch, _PW), jnp.float32),
            pltpu.VMEM((ch, de), jnp.float32),
            pltpu.VMEM((ch, de), jnp.float32),
            pltpu.VMEM((ch, de), jnp.float32),
            pltpu.SemaphoreType.DMA,
        ],
    )
    def sc_edge_pass(p1_hbm, q1_hbm, q2_hbm, snd_hbm, rcv_hbm, ew_hbm,
                     edg_hbm, acc_out, g12_out,
                     acc, sidx, ridx, ewv, edv, prow, yv, q1v, q2v, g12v,
                     sem):
        cid = lax.axis_index("c")
        sid = lax.axis_index("s")
        wid = cid * _NS + sid

        @pl.loop(0, ch)
        def _(r):
            for c in range(_PW // 16):
                yv[r, pl.ds(c * 16, 16)] = jnp.zeros((16,), jnp.float32)

        @pl.loop(0, npt)
        def _(z):
            k = sid + z * _NS

            @pl.when(k < nch)
            def _():
                pltpu.sync_copy(yv, acc.at[pl.ds(k * ch, ch)])

        plsc.subcore_barrier()

        lane = lax.iota(jnp.int32, 16)

        @pl.loop(0, nit)
        def _(j):
            row = wid * nit + j
            base = row * ch
            pltpu.sync_copy(snd_hbm.at[row], sidx)
            pltpu.sync_copy(rcv_hbm.at[row], ridx)
            pltpu.sync_copy(ew_hbm.at[row], ewv)
            pltpu.sync_copy(edg_hbm.at[pl.ds(base, ch)], edv)
            pltpu.async_copy(p1_hbm.at[sidx], prow, sem).wait()
            pltpu.async_copy(q1_hbm.at[sidx], q1v, sem).wait()
            pltpu.async_copy(q2_hbm.at[ridx], q2v, sem).wait()

            for off, lo in ((0, 0), (16, 0), (24, 8)):
                wv16 = ewv[pl.ds(off, 16)]
                for l in range(lo, 16):
                    i = off + l
                    w = wv16[l]
                    wv = jnp.full((16,), w, jnp.float32)
                    for c in range(d // 16):
                        yv[i, pl.ds(c * 16, 16)] = (
                            prow[i, pl.ds(c * 16, 16)] * wv)
                    yv[i, pl.ds(d, 16)] = edv[i, :] * wv
                    tail = jnp.where(lane == 0, w,
                                     jnp.where(lane == 1, 1.0, 0.0))
                    yv[i, pl.ds(d + 16, 16)] = tail
                    g12v[i, :] = q1v[i, :] + q2v[i, :]

            pltpu.sync_copy(yv, acc.at[ridx], add=True)
            pltpu.sync_copy(g12v, g12_out.at[pl.ds(base, ch)])

        plsc.subcore_barrier()

        @pl.loop(0, npt)
        def _(z):
            k = sid + z * _NS

            @pl.when(k < nch)
            def _():
                pltpu.sync_copy(acc.at[pl.ds(k * ch, ch)],
                                acc_out.at[pl.ds(cid * n + k * ch, ch)])

    return sc_edge_pass


def _edge_body(g12_ref, edg_ref, ew_ref, w3e_ref, gve_ref, be_ref, out_ref):
    r = jnp.dot(edg_ref[...], w3e_ref[...], preferred_element_type=jnp.float32)
    out_ref[...] = ew_ref[...] * (g12_ref[...] + r + gve_ref[...]) + be_ref[...]


def _make_node_body(nblocks):
    def _node_body(acca_ref, accb_ref, p2_ref, x_ref, w3n_ref, gvn_ref,
                   bn_ref, glob_ref, wg_ref, bg_ref, wgn_ref, bgn_ref,
                   wge_ref, bge_ref, wfg_ref, bfg_ref,
                   nn_ref, ng_ref, nsum_ref, esum_ref):
        i = pl.program_id(0)
        a = acca_ref[...] + accb_ref[...]
        d = p2_ref.shape[1]
        de = esum_ref.shape[1]
        a128 = a[:, :d]
        e16 = a[:, d:d + de]
        s = a[:, d + de:d + de + 1]
        deg = a[:, d + de + 1:d + de + 2]
        nn_ref[...] = (a128
                       + jnp.dot(e16, w3n_ref[...],
                                 preferred_element_type=jnp.float32)
                       + s * (p2_ref[...] + gvn_ref[...])
                       + deg * bn_ref[...])
        bn_sum = jnp.sum(x_ref[...], axis=0, keepdims=True)
        be_sum = jnp.sum(e16, axis=0, keepdims=True)

        @pl.when(i == 0)
        def _():
            nsum_ref[...] = bn_sum
            esum_ref[...] = be_sum

        @pl.when(i > 0)
        def _():
            nsum_ref[...] += bn_sum
            esum_ref[...] += be_sum

        @pl.when(i == nblocks - 1)
        def _():
            tg = jnp.dot(glob_ref[...], wg_ref[...],
                         preferred_element_type=jnp.float32) + bg_ref[...]
            tn = jnp.dot(nsum_ref[...], wgn_ref[...],
                         preferred_element_type=jnp.float32) + bgn_ref[...]
            te = jnp.dot(esum_ref[...], wge_ref[...],
                         preferred_element_type=jnp.float32) + bge_ref[...]
            fin = jnp.concatenate([tg, tn, te], axis=1)
            ng_ref[...] = jnp.dot(fin, wfg_ref[...],
                                  preferred_element_type=jnp.float32) + bfg_ref[...]

    return _node_body


def kernel(nodes, edges, globals_, senders, receivers, n_node, n_edge,
           edge_weights, W_node, b_node, W_edge, b_edge, W_gn, b_gn,
           W_ge, b_ge, W_g, b_g, W_fg, b_fg):
    n, d = nodes.shape
    e, de = edges.shape
    dg = globals_.shape[1]
    f32 = jnp.float32

    wcat = jnp.concatenate(
        [W_node[:d], W_node[d:2 * d], W_edge[:d], W_edge[d:2 * d]], axis=1)
    w4cat = jnp.concatenate(
        [W_node[2 * d + de:], W_edge[2 * d + de:]], axis=1)
    w3n = W_node[2 * d:2 * d + de]
    w3e = W_edge[2 * d:2 * d + de]

    bn = 2000
    nblocks = n // bn
    p1, p2, q1, q2, gv = pl.pallas_call(
        _pre_body,
        grid=(nblocks,),
        in_specs=[
            pl.BlockSpec((bn, d), lambda i: (i, 0)),
            pl.BlockSpec((1, dg), lambda i: (0, 0)),
            pl.BlockSpec((d, 2 * d + 2 * de), lambda i: (0, 0)),
            pl.BlockSpec((dg, d + de), lambda i: (0, 0)),
        ],
        out_specs=[
            pl.BlockSpec((bn, d), lambda i: (i, 0)),
            pl.BlockSpec((bn, d), lambda i: (i, 0)),
            pl.BlockSpec((bn, de), lambda i: (i, 0)),
            pl.BlockSpec((bn, de), lambda i: (i, 0)),
            pl.BlockSpec((1, d + de), lambda i: (0, 0)),
        ],
        out_shape=[
            jax.ShapeDtypeStruct((n, d), f32),
            jax.ShapeDtypeStruct((n, d), f32),
            jax.ShapeDtypeStruct((n, de), f32),
            jax.ShapeDtypeStruct((n, de), f32),
            jax.ShapeDtypeStruct((1, d + de), f32),
        ],
    )(nodes, globals_, wcat, w4cat)
    gvn = gv[:, :d]
    gve = gv[:, d:]

    nw = _NC * _NS
    ch = 40
    snd2 = senders.astype(jnp.int32).reshape(e // ch, ch)
    rcv2 = receivers.astype(jnp.int32).reshape(e // ch, ch)
    ew2 = edge_weights.astype(f32).reshape(e // ch, ch)
    acc_out, g12 = _make_sc_edge_pass(n, d, e, de)(
        p1, q1, q2, snd2, rcv2, ew2, edges)

    be_blk = 8000
    eblocks = e // be_blk
    new_edges = pl.pallas_call(
        _edge_body,
        grid=(eblocks,),
        in_specs=[
            pl.BlockSpec((be_blk, de), lambda i: (i, 0)),
            pl.BlockSpec((be_blk, de), lambda i: (i, 0)),
            pl.BlockSpec((be_blk, 1), lambda i: (i, 0)),
            pl.BlockSpec((de, de), lambda i: (0, 0)),
            pl.BlockSpec((1, de), lambda i: (0, 0)),
            pl.BlockSpec((1, de), lambda i: (0, 0)),
        ],
        out_specs=pl.BlockSpec((be_blk, de), lambda i: (i, 0)),
        out_shape=jax.ShapeDtypeStruct((e, de), f32),
    )(g12, edges, edge_weights.reshape(e, 1), w3e, gve,
      b_edge.reshape(1, de))

    small = lambda: pl.BlockSpec(None, lambda i: (0, 0))
    new_nodes, new_global = pl.pallas_call(
        _make_node_body(nblocks),
        grid=(nblocks,),
        in_specs=[
            pl.BlockSpec((bn, _PW), lambda i: (i, 0)),
            pl.BlockSpec((bn, _PW), lambda i: (i + nblocks, 0)),
            pl.BlockSpec((bn, d), lambda i: (i, 0)),
            pl.BlockSpec((bn, d), lambda i: (i, 0)),
            pl.BlockSpec((de, d), lambda i: (0, 0)),
            pl.BlockSpec((1, d), lambda i: (0, 0)),
            pl.BlockSpec((1, d), lambda i: (0, 0)),
            pl.BlockSpec((1, dg), lambda i: (0, 0)),
            pl.BlockSpec((dg, dg), lambda i: (0, 0)),
            pl.BlockSpec((1, dg), lambda i: (0, 0)),
            pl.BlockSpec((d, dg), lambda i: (0, 0)),
            pl.BlockSpec((1, dg), lambda i: (0, 0)),
            pl.BlockSpec((de, dg), lambda i: (0, 0)),
            pl.BlockSpec((1, dg), lambda i: (0, 0)),
            pl.BlockSpec((3 * dg, dg), lambda i: (0, 0)),
            pl.BlockSpec((1, dg), lambda i: (0, 0)),
        ],
        out_specs=[
            pl.BlockSpec((bn, d), lambda i: (i, 0)),
            pl.BlockSpec((1, dg), lambda i: (0, 0)),
        ],
        out_shape=[
            jax.ShapeDtypeStruct((n, d), f32),
            jax.ShapeDtypeStruct((1, dg), f32),
        ],
        scratch_shapes=[
            pltpu.VMEM((1, d), f32),
            pltpu.VMEM((1, de), f32),
        ],
    )(acc_out, acc_out, p2, nodes, w3n, gvn, b_node.reshape(1, d),
      globals_, W_g, b_g.reshape(1, dg), W_gn, b_gn.reshape(1, dg),
      W_ge, b_ge.reshape(1, dg), W_fg, b_fg.reshape(1, dg))

    return (new_nodes, new_edges, new_global)


# speedup vs baseline: 2.5681x; 1.6261x over previous
import functools

import jax
import jax.numpy as jnp
from jax import lax
from jax.experimental import pallas as pl
from jax.experimental.pallas import tpu as pltpu
from jax.experimental.pallas import tpu_sc as plsc

_NC = 2
_NS = 16
_PW = 160


def _pre_body(x_ref, g_ref, wcat_ref, w4_ref, t1_ref, p2_ref, q2_ref, gv_ref):
    y = jnp.dot(x_ref[...], wcat_ref[...], preferred_element_type=jnp.float32)
    d = p2_ref.shape[1]
    de = q2_ref.shape[1]
    t1_ref[...] = y[:, :d + de]
    p2_ref[...] = y[:, d + de:2 * d + de]
    q2_ref[...] = y[:, 2 * d + de:]

    @pl.when(pl.program_id(0) == 0)
    def _():
        gv_ref[...] = jnp.dot(g_ref[...], w4_ref[...],
                              preferred_element_type=jnp.float32)


def _make_sc_edge_pass(n, d, e, de):
    nw = _NC * _NS
    epw = e // nw
    ch = 40
    nit = epw // ch
    nch = n // ch
    npt = -(-nch // _NS)
    assert nit % 2 == 0

    mesh = plsc.VectorSubcoreMesh(core_axis_name="c", subcore_axis_name="s",
                                  num_cores=_NC, num_subcores=_NS)

    @functools.partial(
        pl.kernel,
        out_type=(
            jax.ShapeDtypeStruct((_NC * n, _PW), jnp.float32),
            jax.ShapeDtypeStruct((e, de), jnp.float32),
        ),
        mesh=mesh,
        compiler_params=pltpu.CompilerParams(use_tc_tiling_on_sc=False),
        scratch_types=[
            pltpu.VMEM_SHARED((n, _PW), jnp.float32),
            pltpu.VMEM((2, ch), jnp.int32),
            pltpu.VMEM((2, ch), jnp.int32),
            pltpu.VMEM((ch,), jnp.float32),
            pltpu.VMEM((ch,), jnp.float32),
            pltpu.VMEM((ch, d + de), jnp.float32),
            pltpu.VMEM((ch, d + de), jnp.float32),
            pltpu.VMEM((ch, de), jnp.float32),
            pltpu.VMEM((ch, de), jnp.float32),
            pltpu.VMEM((ch, de), jnp.float32),
            pltpu.VMEM((ch, de), jnp.float32),
            pltpu.VMEM((ch, _PW), jnp.float32),
            pltpu.VMEM((ch, de), jnp.float32),
            pltpu.SemaphoreType.DMA,
            pltpu.SemaphoreType.DMA,
        ],
    )
    def sc_edge_pass(t1_hbm, q2_hbm, pk_hbm, ew_hbm, edg_hbm,
                     acc_out, g12_out,
                     acc, pb0, pb1, ew0, ew1, tb0, tb1, qb0, qb1, eb0, eb1,
                     yv, g12v, sem0, sem1):
        cid = lax.axis_index("c")
        sid = lax.axis_index("s")
        wid = cid * _NS + sid
        row0 = wid * nit

        @pl.loop(0, ch)
        def _(r):
            for c in range(_PW // 16):
                yv[r, pl.ds(c * 16, 16)] = jnp.zeros((16,), jnp.float32)

        @pl.loop(0, npt)
        def _(z):
            k = sid + z * _NS

            @pl.when(k < nch)
            def _():
                pltpu.sync_copy(yv, acc.at[pl.ds(k * ch, ch)])

        plsc.subcore_barrier()

        lane = lax.iota(jnp.int32, 16)
        onehot1 = jnp.where(lane == 1, 1.0, 0.0).astype(jnp.float32)

        def load_issue(row, pb, ev, tb, qb, eb, sm):
            pltpu.sync_copy(pk_hbm.at[row], pb)
            pltpu.sync_copy(ew_hbm.at[pl.ds(row * ch, ch)], ev)
            pltpu.async_copy(t1_hbm.at[pb.at[0]], tb, sm)
            pltpu.async_copy(q2_hbm.at[pb.at[1]], qb, sm)
            pltpu.async_copy(edg_hbm.at[pl.ds(row * ch, ch)], eb, sm)

        def drain(row, pb, tb, qb, eb, sm):
            pltpu.make_async_copy(t1_hbm.at[pb.at[0]], tb, sm).wait()
            pltpu.make_async_copy(q2_hbm.at[pb.at[1]], qb, sm).wait()
            pltpu.make_async_copy(edg_hbm.at[pl.ds(row * ch, ch)], eb,
                                  sm).wait()

        def combine(row, pb, ev, tb, qb, eb):
            for off, lo in ((0, 0), (16, 0), (24, 8)):
                wv16 = ev[pl.ds(off, 16)]
                for l in range(lo, 16):
                    i = off + l
                    w = wv16[l]
                    wv = jnp.full((16,), w, jnp.float32)
                    for c in range(d // 16):
                        yv[i, pl.ds(c * 16, 16)] = (
                            tb[i, pl.ds(c * 16, 16)] * wv)
                    yv[i, pl.ds(d, 16)] = eb[i, :] * wv
                    yv[i, pl.ds(d + 16, 16)] = jnp.where(lane == 0, w,
                                                         onehot1)
                    g12v[i, :] = tb[i, pl.ds(d, 16)] + qb[i, :]
            pltpu.sync_copy(yv, acc.at[pb.at[1]], add=True)
            pltpu.sync_copy(g12v, g12_out.at[pl.ds(row * ch, ch)])

        load_issue(row0, pb0, ew0, tb0, qb0, eb0, sem0)

        @pl.loop(0, nit, step=2)
        def _(j):
            ra = row0 + j
            rb = ra + 1
            load_issue(rb, pb1, ew1, tb1, qb1, eb1, sem1)
            drain(ra, pb0, tb0, qb0, eb0, sem0)
            combine(ra, pb0, ew0, tb0, qb0, eb0)
            rn = jnp.where(j + 2 >= nit, row0, ra + 2)
            load_issue(rn, pb0, ew0, tb0, qb0, eb0, sem0)
            drain(rb, pb1, tb1, qb1, eb1, sem1)
            combine(rb, pb1, ew1, tb1, qb1, eb1)

        drain(row0, pb0, tb0, qb0, eb0, sem0)
        plsc.subcore_barrier()

        @pl.loop(0, npt)
        def _(z):
            k = sid + z * _NS

            @pl.when(k < nch)
            def _():
                pltpu.sync_copy(acc.at[pl.ds(k * ch, ch)],
                                acc_out.at[pl.ds(cid * n + k * ch, ch)])

    return sc_edge_pass


def _edge_body(g12_ref, edg_ref, ew_ref, w3e_ref, gve_ref, be_ref, out_ref):
    r = jnp.dot(edg_ref[...], w3e_ref[...], preferred_element_type=jnp.float32)
    out_ref[...] = ew_ref[...] * (g12_ref[...] + r + gve_ref[...]) + be_ref[...]


def _make_node_body(nblocks):
    def _node_body(acca_ref, accb_ref, p2_ref, x_ref, w3n_ref, gvn_ref,
                   bn_ref, glob_ref, wg_ref, bg_ref, wgn_ref, bgn_ref,
                   wge_ref, bge_ref, wfg_ref, bfg_ref,
                   nn_ref, ng_ref, nsum_ref, esum_ref):
        i = pl.program_id(0)
        a = acca_ref[...] + accb_ref[...]
        d = p2_ref.shape[1]
        de = esum_ref.shape[1]
        a128 = a[:, :d]
        e16 = a[:, d:d + de]
        s = a[:, d + de:d + de + 1]
        deg = a[:, d + de + 1:d + de + 2]
        nn_ref[...] = (a128
                       + jnp.dot(e16, w3n_ref[...],
                                 preferred_element_type=jnp.float32)
                       + s * (p2_ref[...] + gvn_ref[...])
                       + deg * bn_ref[...])
        bn_sum = jnp.sum(x_ref[...], axis=0, keepdims=True)
        be_sum = jnp.sum(e16, axis=0, keepdims=True)

        @pl.when(i == 0)
        def _():
            nsum_ref[...] = bn_sum
            esum_ref[...] = be_sum

        @pl.when(i > 0)
        def _():
            nsum_ref[...] += bn_sum
            esum_ref[...] += be_sum

        @pl.when(i == nblocks - 1)
        def _():
            tg = jnp.dot(glob_ref[...], wg_ref[...],
                         preferred_element_type=jnp.float32) + bg_ref[...]
            tn = jnp.dot(nsum_ref[...], wgn_ref[...],
                         preferred_element_type=jnp.float32) + bgn_ref[...]
            te = jnp.dot(esum_ref[...], wge_ref[...],
                         preferred_element_type=jnp.float32) + bge_ref[...]
            fin = jnp.concatenate([tg, tn, te], axis=1)
            ng_ref[...] = jnp.dot(fin, wfg_ref[...],
                                  preferred_element_type=jnp.float32) + bfg_ref[...]

    return _node_body


def kernel(nodes, edges, globals_, senders, receivers, n_node, n_edge,
           edge_weights, W_node, b_node, W_edge, b_edge, W_gn, b_gn,
           W_ge, b_ge, W_g, b_g, W_fg, b_fg):
    n, d = nodes.shape
    e, de = edges.shape
    dg = globals_.shape[1]
    f32 = jnp.float32

    wcat = jnp.concatenate(
        [W_node[:d], W_edge[:d], W_node[d:2 * d], W_edge[d:2 * d]], axis=1)
    w4cat = jnp.concatenate(
        [W_node[2 * d + de:], W_edge[2 * d + de:]], axis=1)
    w3n = W_node[2 * d:2 * d + de]
    w3e = W_edge[2 * d:2 * d + de]

    bn = 2000
    nblocks = n // bn
    t1, p2, q2, gv = pl.pallas_call(
        _pre_body,
        grid=(nblocks,),
        in_specs=[
            pl.BlockSpec((bn, d), lambda i: (i, 0)),
            pl.BlockSpec((1, dg), lambda i: (0, 0)),
            pl.BlockSpec((d, 2 * d + 2 * de), lambda i: (0, 0)),
            pl.BlockSpec((dg, d + de), lambda i: (0, 0)),
        ],
        out_specs=[
            pl.BlockSpec((bn, d + de), lambda i: (i, 0)),
            pl.BlockSpec((bn, d), lambda i: (i, 0)),
            pl.BlockSpec((bn, de), lambda i: (i, 0)),
            pl.BlockSpec((1, d + de), lambda i: (0, 0)),
        ],
        out_shape=[
            jax.ShapeDtypeStruct((n, d + de), f32),
            jax.ShapeDtypeStruct((n, d), f32),
            jax.ShapeDtypeStruct((n, de), f32),
            jax.ShapeDtypeStruct((1, d + de), f32),
        ],
    )(nodes, globals_, wcat, w4cat)
    gvn = gv[:, :d]
    gve = gv[:, d:]

    ch = 40
    snd2 = senders.astype(jnp.int32).reshape(e // ch, ch)
    rcv2 = receivers.astype(jnp.int32).reshape(e // ch, ch)
    pk = jnp.stack([snd2, rcv2], axis=1)
    acc_out, g12 = _make_sc_edge_pass(n, d, e, de)(
        t1, q2, pk, edge_weights.astype(f32), edges)

    be_blk = 8000
    eblocks = e // be_blk
    new_edges = pl.pallas_call(
        _edge_body,
        grid=(eblocks,),
        in_specs=[
            pl.BlockSpec((be_blk, de), lambda i: (i, 0)),
            pl.BlockSpec((be_blk, de), lambda i: (i, 0)),
            pl.BlockSpec((be_blk, 1), lambda i: (i, 0)),
            pl.BlockSpec((de, de), lambda i: (0, 0)),
            pl.BlockSpec((1, de), lambda i: (0, 0)),
            pl.BlockSpec((1, de), lambda i: (0, 0)),
        ],
        out_specs=pl.BlockSpec((be_blk, de), lambda i: (i, 0)),
        out_shape=jax.ShapeDtypeStruct((e, de), f32),
    )(g12, edges, edge_weights.reshape(e, 1), w3e, gve,
      b_edge.reshape(1, de))

    new_nodes, new_global = pl.pallas_call(
        _make_node_body(nblocks),
        grid=(nblocks,),
        in_specs=[
            pl.BlockSpec((bn, _PW), lambda i: (i, 0)),
            pl.BlockSpec((bn, _PW), lambda i: (i + nblocks, 0)),
            pl.BlockSpec((bn, d), lambda i: (i, 0)),
            pl.BlockSpec((bn, d), lambda i: (i, 0)),
            pl.BlockSpec((de, d), lambda i: (0, 0)),
            pl.BlockSpec((1, d), lambda i: (0, 0)),
            pl.BlockSpec((1, d), lambda i: (0, 0)),
            pl.BlockSpec((1, dg), lambda i: (0, 0)),
            pl.BlockSpec((dg, dg), lambda i: (0, 0)),
            pl.BlockSpec((1, dg), lambda i: (0, 0)),
            pl.BlockSpec((d, dg), lambda i: (0, 0)),
            pl.BlockSpec((1, dg), lambda i: (0, 0)),
            pl.BlockSpec((de, dg), lambda i: (0, 0)),
            pl.BlockSpec((1, dg), lambda i: (0, 0)),
            pl.BlockSpec((3 * dg, dg), lambda i: (0, 0)),
            pl.BlockSpec((1, dg), lambda i: (0, 0)),
        ],
        out_specs=[
            pl.BlockSpec((bn, d), lambda i: (i, 0)),
            pl.BlockSpec((1, dg), lambda i: (0, 0)),
        ],
        out_shape=[
            jax.ShapeDtypeStruct((n, d), f32),
            jax.ShapeDtypeStruct((1, dg), f32),
        ],
        scratch_shapes=[
            pltpu.VMEM((1, d), f32),
            pltpu.VMEM((1, de), f32),
        ],
    )(acc_out, acc_out, p2, nodes, w3n, gvn, b_node.reshape(1, d),
      globals_, W_g, b_g.reshape(1, dg), W_gn, b_gn.reshape(1, dg),
      W_ge, b_ge.reshape(1, dg), W_fg, b_fg.reshape(1, dg))

    return (new_nodes, new_edges, new_global)


# speedup vs baseline: 3.2729x; 1.2744x over previous
import functools

import jax
import jax.numpy as jnp
from jax import lax
from jax.experimental import pallas as pl
from jax.experimental.pallas import tpu as pltpu
from jax.experimental.pallas import tpu_sc as plsc

_NC = 2
_NS = 16
_PW = 160


def _pre_body(x_ref, g_ref, wcat_ref, w4_ref, t1_ref, p2_ref, q2_ref, gv_ref):
    y = jnp.dot(x_ref[...], wcat_ref[...], preferred_element_type=jnp.float32)
    d = p2_ref.shape[1]
    de = q2_ref.shape[1]
    t1_ref[...] = y[:, :d + de]
    p2_ref[...] = y[:, d + de:2 * d + de]
    q2_ref[...] = y[:, 2 * d + de:]

    @pl.when(pl.program_id(0) == 0)
    def _():
        gv_ref[...] = jnp.dot(g_ref[...], w4_ref[...],
                              preferred_element_type=jnp.float32)


def _make_sc_edge_pass(n, d, e, de):
    nw = _NC * _NS
    epw = e // nw
    ch = 40
    nit = epw // ch
    nch = n // ch
    npt = -(-nch // _NS)
    assert nit % 2 == 0

    mesh = plsc.VectorSubcoreMesh(core_axis_name="c", subcore_axis_name="s",
                                  num_cores=_NC, num_subcores=_NS)

    rpc = ch * de // 128

    @functools.partial(
        pl.kernel,
        out_type=(
            jax.ShapeDtypeStruct((_NC * n, d), jnp.float32),
            jax.ShapeDtypeStruct((_NC * n, 2 * de), jnp.float32),
            jax.ShapeDtypeStruct((e // 8, 128), jnp.float32),
            jax.ShapeDtypeStruct((e // 8, 128), jnp.float32),
        ),
        mesh=mesh,
        compiler_params=pltpu.CompilerParams(use_tc_tiling_on_sc=False),
        scratch_types=[
            pltpu.VMEM_SHARED((n, _PW), jnp.float32),
            pltpu.VMEM((ch,), jnp.int32),
            pltpu.VMEM((ch,), jnp.int32),
            pltpu.VMEM((ch,), jnp.int32),
            pltpu.VMEM((ch,), jnp.int32),
            pltpu.VMEM((ch,), jnp.float32),
            pltpu.VMEM((ch,), jnp.float32),
            pltpu.VMEM((ch, d + de), jnp.float32),
            pltpu.VMEM((ch, d + de), jnp.float32),
            pltpu.VMEM((ch, de), jnp.float32),
            pltpu.VMEM((ch, de), jnp.float32),
            pltpu.VMEM((rpc, 128), jnp.float32),
            pltpu.VMEM((rpc, 128), jnp.float32),
            pltpu.VMEM((ch, _PW), jnp.float32),
            pltpu.VMEM((rpc, 128), jnp.float32),
            pltpu.VMEM((rpc, 128), jnp.float32),
            pltpu.VMEM((de,), jnp.float32),
            pltpu.SemaphoreType.DMA,
            pltpu.SemaphoreType.DMA,
        ],
    )
    def sc_edge_pass(t1_hbm, q2_hbm, snd_hbm, rcv_hbm, ew_hbm, ed8_hbm,
                     gve_hbm,
                     a128_out, a32_out, ne1_out, ne2_out,
                     acc, sb0, sb1, rb0, rb1, ew0, ew1, tb0, tb1, qb0, qb1,
                     eb0, eb1, yv, ne1v, ne2v, gvb, sem0, sem1):
        cid = lax.axis_index("c")
        sid = lax.axis_index("s")
        wid = cid * _NS + sid
        row0 = wid * nit

        @pl.loop(0, ch)
        def _(r):
            for c in range(_PW // 16):
                yv[r, pl.ds(c * 16, 16)] = jnp.zeros((16,), jnp.float32)

        @pl.loop(0, npt)
        def _(z):
            k = sid + z * _NS

            @pl.when(k < nch)
            def _():
                pltpu.sync_copy(yv, acc.at[pl.ds(k * ch, ch)])

        plsc.subcore_barrier()

        lane = lax.iota(jnp.int32, 16)
        onehot1 = jnp.where(lane == 1, 1.0, 0.0).astype(jnp.float32)
        pltpu.sync_copy(gve_hbm, gvb)
        gvec = gvb[...]

        def load_issue(row, sb, rb, ev, tb, qb, eb, sm):
            pltpu.sync_copy(snd_hbm.at[pl.ds(row * ch, ch)], sb)
            pltpu.sync_copy(rcv_hbm.at[pl.ds(row * ch, ch)], rb)
            pltpu.sync_copy(ew_hbm.at[pl.ds(row * ch, ch)], ev)
            pltpu.async_copy(t1_hbm.at[sb], tb, sm)
            pltpu.async_copy(q2_hbm.at[rb], qb, sm)
            pltpu.async_copy(ed8_hbm.at[pl.ds(row * rpc, rpc)], eb, sm)

        def drain(row, sb, rb, tb, qb, eb, sm):
            pltpu.make_async_copy(t1_hbm.at[sb], tb, sm).wait()
            pltpu.make_async_copy(q2_hbm.at[rb], qb, sm).wait()
            pltpu.make_async_copy(ed8_hbm.at[pl.ds(row * rpc, rpc)], eb,
                                  sm).wait()

        def combine(row, rb, ev, tb, qb, eb):
            for off, lo in ((0, 0), (16, 0), (24, 8)):
                wv16 = ev[pl.ds(off, 16)]
                for l in range(lo, 16):
                    i = off + l
                    w = wv16[l]
                    wv = jnp.full((16,), w, jnp.float32)
                    for c in range(d // 16):
                        yv[i, pl.ds(c * 16, 16)] = (
                            tb[i, pl.ds(c * 16, 16)] * wv)
                    ewe = eb[i // 8, pl.ds((i % 8) * 16, 16)] * wv
                    yv[i, pl.ds(d, 16)] = ewe
                    ne2v[i // 8, pl.ds((i % 8) * 16, 16)] = ewe
                    yv[i, pl.ds(d + 16, 16)] = jnp.where(lane == 0, w,
                                                         onehot1)
                    ne1v[i // 8, pl.ds((i % 8) * 16, 16)] = (
                        (tb[i, pl.ds(d, 16)] + qb[i, :] + gvec) * wv)
            pltpu.sync_copy(yv, acc.at[rb], add=True)
            pltpu.sync_copy(ne1v, ne1_out.at[pl.ds(row * rpc, rpc)])
            pltpu.sync_copy(ne2v, ne2_out.at[pl.ds(row * rpc, rpc)])

        load_issue(row0, sb0, rb0, ew0, tb0, qb0, eb0, sem0)

        @pl.loop(0, nit, step=2)
        def _(j):
            ra = row0 + j
            rb_ = ra + 1
            load_issue(rb_, sb1, rb1, ew1, tb1, qb1, eb1, sem1)
            drain(ra, sb0, rb0, tb0, qb0, eb0, sem0)
            combine(ra, rb0, ew0, tb0, qb0, eb0)
            rn = jnp.where(j + 2 >= nit, row0, ra + 2)
            load_issue(rn, sb0, rb0, ew0, tb0, qb0, eb0, sem0)
            drain(rb_, sb1, rb1, tb1, qb1, eb1, sem1)
            combine(rb_, rb1, ew1, tb1, qb1, eb1)

        drain(row0, sb0, rb0, tb0, qb0, eb0, sem0)
        plsc.subcore_barrier()

        @pl.loop(0, npt)
        def _(z):
            k = sid + z * _NS

            @pl.when(k < nch)
            def _():
                pltpu.sync_copy(acc.at[pl.ds(k * ch, ch), pl.ds(0, d)],
                                a128_out.at[pl.ds(cid * n + k * ch, ch)])
                pltpu.sync_copy(acc.at[pl.ds(k * ch, ch), pl.ds(d, 2 * de)],
                                a32_out.at[pl.ds(cid * n + k * ch, ch)])

    return sc_edge_pass


def _edge_body(ne1_ref, ne2_ref, wbd_ref, bet_ref, out_ref):
    r = jnp.dot(ne2_ref[...], wbd_ref[...], preferred_element_type=jnp.float32)
    out_ref[...] = ne1_ref[...] + r + bet_ref[...]


def _make_node_body(nblocks):
    def _node_body(acca_ref, accb_ref, ta_ref, tb_ref, p2_ref, x_ref,
                   w3n_ref, gvn_ref,
                   bn_ref, glob_ref, wg_ref, bg_ref, wgn_ref, bgn_ref,
                   wge_ref, bge_ref, wfg_ref, bfg_ref,
                   nn_ref, ng_ref, nsum_ref, esum_ref):
        i = pl.program_id(0)
        a128 = acca_ref[...] + accb_ref[...]
        t = ta_ref[...] + tb_ref[...]
        d = p2_ref.shape[1]
        de = esum_ref.shape[1]
        e16 = t[:, :de]
        s = t[:, de:de + 1]
        deg = t[:, de + 1:de + 2]
        nn_ref[...] = (a128
                       + jnp.dot(e16, w3n_ref[...],
                                 preferred_element_type=jnp.float32)
                       + s * (p2_ref[...] + gvn_ref[...])
                       + deg * bn_ref[...])
        bn_sum = jnp.sum(x_ref[...], axis=0, keepdims=True)
        be_sum = jnp.sum(e16, axis=0, keepdims=True)

        @pl.when(i == 0)
        def _():
            nsum_ref[...] = bn_sum
            esum_ref[...] = be_sum

        @pl.when(i > 0)
        def _():
            nsum_ref[...] += bn_sum
            esum_ref[...] += be_sum

        @pl.when(i == nblocks - 1)
        def _():
            tg = jnp.dot(glob_ref[...], wg_ref[...],
                         preferred_element_type=jnp.float32) + bg_ref[...]
            tn = jnp.dot(nsum_ref[...], wgn_ref[...],
                         preferred_element_type=jnp.float32) + bgn_ref[...]
            te = jnp.dot(esum_ref[...], wge_ref[...],
                         preferred_element_type=jnp.float32) + bge_ref[...]
            fin = jnp.concatenate([tg, tn, te], axis=1)
            ng_ref[...] = jnp.dot(fin, wfg_ref[...],
                                  preferred_element_type=jnp.float32) + bfg_ref[...]

    return _node_body


def kernel(nodes, edges, globals_, senders, receivers, n_node, n_edge,
           edge_weights, W_node, b_node, W_edge, b_edge, W_gn, b_gn,
           W_ge, b_ge, W_g, b_g, W_fg, b_fg):
    n, d = nodes.shape
    e, de = edges.shape
    dg = globals_.shape[1]
    f32 = jnp.float32

    wcat = jnp.concatenate(
        [W_node[:d], W_edge[:d], W_node[d:2 * d], W_edge[d:2 * d]], axis=1)
    w4cat = jnp.concatenate(
        [W_node[2 * d + de:], W_edge[2 * d + de:]], axis=1)
    w3n = W_node[2 * d:2 * d + de]
    w3e = W_edge[2 * d:2 * d + de]

    bn = 2000
    nblocks = n // bn
    t1, p2, q2, gv = pl.pallas_call(
        _pre_body,
        grid=(nblocks,),
        in_specs=[
            pl.BlockSpec((bn, d), lambda i: (i, 0)),
            pl.BlockSpec((1, dg), lambda i: (0, 0)),
            pl.BlockSpec((d, 2 * d + 2 * de), lambda i: (0, 0)),
            pl.BlockSpec((dg, d + de), lambda i: (0, 0)),
        ],
        out_specs=[
            pl.BlockSpec((bn, d + de), lambda i: (i, 0)),
            pl.BlockSpec((bn, d), lambda i: (i, 0)),
            pl.BlockSpec((bn, de), lambda i: (i, 0)),
            pl.BlockSpec((1, d + de), lambda i: (0, 0)),
        ],
        out_shape=[
            jax.ShapeDtypeStruct((n, d + de), f32),
            jax.ShapeDtypeStruct((n, d), f32),
            jax.ShapeDtypeStruct((n, de), f32),
            jax.ShapeDtypeStruct((1, d + de), f32),
        ],
    )(nodes, globals_, wcat, w4cat)
    gvn = gv[:, :d]
    gve1 = gv[0, d:]

    ed8 = edges.reshape(e // 8, (8 * de))
    a128, a32, ne1, ne2 = _make_sc_edge_pass(n, d, e, de)(
        t1, q2, senders.astype(jnp.int32), receivers.astype(jnp.int32),
        edge_weights.astype(f32), ed8, gve1)

    wbd = jnp.kron(jnp.eye(8, dtype=f32), w3e)
    bet = jnp.tile(b_edge, 8).reshape(1, 8 * de)
    ep_blk = 5000
    eblocks = (e // 8) // ep_blk
    ne_packed = pl.pallas_call(
        _edge_body,
        grid=(eblocks,),
        in_specs=[
            pl.BlockSpec((ep_blk, 8 * de), lambda i: (i, 0)),
            pl.BlockSpec((ep_blk, 8 * de), lambda i: (i, 0)),
            pl.BlockSpec((8 * de, 8 * de), lambda i: (0, 0)),
            pl.BlockSpec((1, 8 * de), lambda i: (0, 0)),
        ],
        out_specs=pl.BlockSpec((ep_blk, 8 * de), lambda i: (i, 0)),
        out_shape=jax.ShapeDtypeStruct((e // 8, 8 * de), f32),
    )(ne1, ne2, wbd, bet)
    new_edges = ne_packed.reshape(e, de)

    new_nodes, new_global = pl.pallas_call(
        _make_node_body(nblocks),
        grid=(nblocks,),
        in_specs=[
            pl.BlockSpec((bn, d), lambda i: (i, 0)),
            pl.BlockSpec((bn, d), lambda i: (i + nblocks, 0)),
            pl.BlockSpec((bn, 2 * de), lambda i: (i, 0)),
            pl.BlockSpec((bn, 2 * de), lambda i: (i + nblocks, 0)),
            pl.BlockSpec((bn, d), lambda i: (i, 0)),
            pl.BlockSpec((bn, d), lambda i: (i, 0)),
            pl.BlockSpec((de, d), lambda i: (0, 0)),
            pl.BlockSpec((1, d), lambda i: (0, 0)),
            pl.BlockSpec((1, d), lambda i: (0, 0)),
            pl.BlockSpec((1, dg), lambda i: (0, 0)),
            pl.BlockSpec((dg, dg), lambda i: (0, 0)),
            pl.BlockSpec((1, dg), lambda i: (0, 0)),
            pl.BlockSpec((d, dg), lambda i: (0, 0)),
            pl.BlockSpec((1, dg), lambda i: (0, 0)),
            pl.BlockSpec((de, dg), lambda i: (0, 0)),
            pl.BlockSpec((1, dg), lambda i: (0, 0)),
            pl.BlockSpec((3 * dg, dg), lambda i: (0, 0)),
            pl.BlockSpec((1, dg), lambda i: (0, 0)),
        ],
        out_specs=[
            pl.BlockSpec((bn, d), lambda i: (i, 0)),
            pl.BlockSpec((1, dg), lambda i: (0, 0)),
        ],
        out_shape=[
            jax.ShapeDtypeStruct((n, d), f32),
            jax.ShapeDtypeStruct((1, dg), f32),
        ],
        scratch_shapes=[
            pltpu.VMEM((1, d), f32),
            pltpu.VMEM((1, de), f32),
        ],
    )(a128, a128, a32, a32, p2, nodes, w3n, gvn, b_node.reshape(1, d),
      globals_, W_g, b_g.reshape(1, dg), W_gn, b_gn.reshape(1, dg),
      W_ge, b_ge.reshape(1, dg), W_fg, b_fg.reshape(1, dg))

    return (new_nodes, new_edges, new_global)


# speedup vs baseline: 3.7817x; 1.1555x over previous
import functools

import jax
import jax.numpy as jnp
from jax import lax
from jax.experimental import pallas as pl
from jax.experimental.pallas import tpu as pltpu
from jax.experimental.pallas import tpu_sc as plsc

_NC = 2
_NS = 16
_PW = 160


def _pre_body(x_ref, g_ref, wcat_ref, w4_ref, t1_ref, p2_ref, q2_ref, gv_ref):
    y = jnp.dot(x_ref[...], wcat_ref[...], preferred_element_type=jnp.float32)
    d = p2_ref.shape[1]
    de = q2_ref.shape[1]
    t1_ref[...] = y[:, :d + de]
    p2_ref[...] = y[:, d + de:2 * d + de]
    q2_ref[...] = y[:, 2 * d + de:]

    @pl.when(pl.program_id(0) == 0)
    def _():
        gv_ref[...] = jnp.dot(g_ref[...], w4_ref[...],
                              preferred_element_type=jnp.float32)


def _make_sc_edge_pass(n, d, e, de):
    nw = _NC * _NS
    epw = e // nw
    ch = 40
    nit = epw // ch
    nch = n // ch
    npt = -(-nch // _NS)
    assert nit % 2 == 0

    mesh = plsc.VectorSubcoreMesh(core_axis_name="c", subcore_axis_name="s",
                                  num_cores=_NC, num_subcores=_NS)

    rpc = ch * de // 128

    @functools.partial(
        pl.kernel,
    out_type=(
            jax.ShapeDtypeStruct((_NC * n, d), jnp.float32),
            jax.ShapeDtypeStruct((_NC * n, 128), jnp.float32),
            jax.ShapeDtypeStruct((2 * (e // 8), 128), jnp.float32),
        ),
        mesh=mesh,
        compiler_params=pltpu.CompilerParams(use_tc_tiling_on_sc=False),
        scratch_types=[
            pltpu.VMEM_SHARED((n, _PW), jnp.float32),
            pltpu.VMEM((3 * ch,), jnp.int32),
            pltpu.VMEM((3 * ch,), jnp.int32),
            pltpu.VMEM((ch,), jnp.int32),
            pltpu.VMEM((ch, d + de), jnp.float32),
            pltpu.VMEM((ch, d + de), jnp.float32),
            pltpu.VMEM((ch, de), jnp.float32),
            pltpu.VMEM((ch, de), jnp.float32),
            pltpu.VMEM((rpc, 128), jnp.float32),
            pltpu.VMEM((rpc, 128), jnp.float32),
            pltpu.VMEM((ch, _PW), jnp.float32),
            pltpu.VMEM((2 * rpc, 128), jnp.float32),
            pltpu.VMEM((de,), jnp.float32),
            pltpu.SemaphoreType.DMA,
            pltpu.SemaphoreType.DMA,
        ],
    )
    def sc_edge_pass(t1_hbm, q2_hbm, pk_hbm, ed8_hbm, gve_hbm,
                     a128_out, a32_out, ne_out,
                     acc, pb0, pb1, rbuf, tb0, tb1, qb0, qb1,
                     eb0, eb1, yv, nv, gvb, sem0, sem1):
        cid = lax.axis_index("c")
        sid = lax.axis_index("s")
        wid = cid * _NS + sid
        row0 = wid * nit

        @pl.loop(0, ch)
        def _(r):
            for c in range(_PW // 16):
                yv[r, pl.ds(c * 16, 16)] = jnp.zeros((16,), jnp.float32)

        @pl.loop(0, npt)
        def _(z):
            k = sid + z * _NS

            @pl.when(k < nch)
            def _():
                pltpu.sync_copy(yv, acc.at[pl.ds(k * ch, ch)])

        plsc.subcore_barrier()

        lane = lax.iota(jnp.int32, 16)
        onehot1 = jnp.where(lane == 1, 1.0, 0.0).astype(jnp.float32)
        pltpu.sync_copy(gve_hbm, gvb)
        gvec = gvb[...]

        def load_issue(row, pb, tb, qb, eb, sm):
            pltpu.sync_copy(pk_hbm.at[pl.ds(row * 3 * ch, 3 * ch)], pb)
            pltpu.async_copy(t1_hbm.at[pb.at[pl.ds(0, ch)]], tb, sm)
            pltpu.async_copy(q2_hbm.at[pb.at[pl.ds(ch, ch)]], qb, sm)
            pltpu.async_copy(ed8_hbm.at[pl.ds(row * rpc, rpc)], eb, sm)

        def drain(row, pb, tb, qb, eb, sm):
            pltpu.make_async_copy(t1_hbm.at[pb.at[pl.ds(0, ch)]], tb,
                                  sm).wait()
            pltpu.make_async_copy(q2_hbm.at[pb.at[pl.ds(ch, ch)]], qb,
                                  sm).wait()
            pltpu.make_async_copy(ed8_hbm.at[pl.ds(row * rpc, rpc)], eb,
                                  sm).wait()

        def combine(row, pb, tb, qb, eb):
            rbuf[pl.ds(0, 16)] = pb[pl.ds(ch, 16)]
            rbuf[pl.ds(16, 16)] = pb[pl.ds(ch + 16, 16)]
            rbuf[pl.ds(24, 16)] = pb[pl.ds(ch + 24, 16)]
            for off, lo in ((0, 0), (16, 0), (24, 8)):
                wv16 = lax.bitcast_convert_type(pb[pl.ds(2 * ch + off, 16)],
                                                jnp.float32)
                for l in range(lo, 16):
                    i = off + l
                    w = wv16[l]
                    wv = jnp.full((16,), w, jnp.float32)
                    for c in range(d // 16):
                        yv[i, pl.ds(c * 16, 16)] = (
                            tb[i, pl.ds(c * 16, 16)] * wv)
                    ewe = eb[i // 8, pl.ds((i % 8) * 16, 16)] * wv
                    yv[i, pl.ds(d, 16)] = ewe
                    nv[rpc + i // 8, pl.ds((i % 8) * 16, 16)] = ewe
                    yv[i, pl.ds(d + 16, 16)] = jnp.where(lane == 0, w,
                                                         onehot1)
                    nv[i // 8, pl.ds((i % 8) * 16, 16)] = (
                        (tb[i, pl.ds(d, 16)] + qb[i, :] + gvec) * wv)
            pltpu.sync_copy(yv, acc.at[rbuf], add=True)
            pltpu.sync_copy(nv, ne_out.at[pl.ds(row * 2 * rpc, 2 * rpc)])

        load_issue(row0, pb0, tb0, qb0, eb0, sem0)

        @pl.loop(0, nit, step=2)
        def _(j):
            ra = row0 + j
            rb_ = ra + 1
            load_issue(rb_, pb1, tb1, qb1, eb1, sem1)
            drain(ra, pb0, tb0, qb0, eb0, sem0)
            combine(ra, pb0, tb0, qb0, eb0)
            rn = jnp.where(j + 2 >= nit, row0, ra + 2)
            load_issue(rn, pb0, tb0, qb0, eb0, sem0)
            drain(rb_, pb1, tb1, qb1, eb1, sem1)
            combine(rb_, pb1, tb1, qb1, eb1)

        drain(row0, pb0, tb0, qb0, eb0, sem0)
        plsc.subcore_barrier()

        rpt = n // _NS
        r0 = sid * rpt
        pltpu.sync_copy(acc.at[pl.ds(r0, rpt), pl.ds(0, d)],
                        a128_out.at[pl.ds(cid * n + r0, rpt)])
        pltpu.sync_copy(acc.at[pl.ds(r0, rpt), pl.ds(d, 2 * de)],
                        a32_out.at[pl.ds(cid * n + r0, rpt),
                                   pl.ds(0, 2 * de)])

    return sc_edge_pass


def _make_edge_body(rpc):
    def _edge_body(ne_ref, wbd_ref, bet_ref, out_ref):
        x = ne_ref[...]
        g = x.reshape(-1, 2, rpc, x.shape[-1])
        ne1 = g[:, 0].reshape(-1, x.shape[-1])
        ne2 = g[:, 1].reshape(-1, x.shape[-1])
        r = jnp.dot(ne2, wbd_ref[...], preferred_element_type=jnp.float32)
        out_ref[...] = ne1 + r + bet_ref[...]

    return _edge_body


def _make_node_body(nblocks):
    def _node_body(acca_ref, accb_ref, ta_ref, tb_ref, p2_ref, x_ref,
                   w3n_ref, gvn_ref,
                   bn_ref, glob_ref, wg_ref, bg_ref, wgn_ref, bgn_ref,
                   wge_ref, bge_ref, wfg_ref, bfg_ref,
                   nn_ref, ng_ref, nsum_ref, esum_ref):
        i = pl.program_id(0)
        a128 = acca_ref[...] + accb_ref[...]
        t = ta_ref[...] + tb_ref[...]
        d = p2_ref.shape[1]
        de = esum_ref.shape[1]
        e16 = t[:, :de]
        s = t[:, de:de + 1]
        deg = t[:, de + 1:de + 2]
        nn_ref[...] = (a128
                       + jnp.dot(e16, w3n_ref[...],
                                 preferred_element_type=jnp.float32)
                       + s * (p2_ref[...] + gvn_ref[...])
                       + deg * bn_ref[...])
        bn_sum = jnp.sum(x_ref[...], axis=0, keepdims=True)
        be_sum = jnp.sum(e16, axis=0, keepdims=True)

        @pl.when(i == 0)
        def _():
            nsum_ref[...] = bn_sum
            esum_ref[...] = be_sum

        @pl.when(i > 0)
        def _():
            nsum_ref[...] += bn_sum
            esum_ref[...] += be_sum

        @pl.when(i == nblocks - 1)
        def _():
            tg = jnp.dot(glob_ref[...], wg_ref[...],
                         preferred_element_type=jnp.float32) + bg_ref[...]
            tn = jnp.dot(nsum_ref[...], wgn_ref[...],
                         preferred_element_type=jnp.float32) + bgn_ref[...]
            te = jnp.dot(esum_ref[...], wge_ref[...],
                         preferred_element_type=jnp.float32) + bge_ref[...]
            fin = jnp.concatenate([tg, tn, te], axis=1)
            ng_ref[...] = jnp.dot(fin, wfg_ref[...],
                                  preferred_element_type=jnp.float32) + bfg_ref[...]

    return _node_body


def kernel(nodes, edges, globals_, senders, receivers, n_node, n_edge,
           edge_weights, W_node, b_node, W_edge, b_edge, W_gn, b_gn,
           W_ge, b_ge, W_g, b_g, W_fg, b_fg):
    n, d = nodes.shape
    e, de = edges.shape
    dg = globals_.shape[1]
    f32 = jnp.float32

    wcat = jnp.concatenate(
        [W_node[:d], W_edge[:d], W_node[d:2 * d], W_edge[d:2 * d]], axis=1)
    w4cat = jnp.concatenate(
        [W_node[2 * d + de:], W_edge[2 * d + de:]], axis=1)
    w3n = W_node[2 * d:2 * d + de]
    w3e = W_edge[2 * d:2 * d + de]

    bn = 2000
    nblocks = n // bn
    t1, p2, q2, gv = pl.pallas_call(
        _pre_body,
        grid=(nblocks,),
        in_specs=[
            pl.BlockSpec((bn, d), lambda i: (i, 0)),
            pl.BlockSpec((1, dg), lambda i: (0, 0)),
            pl.BlockSpec((d, 2 * d + 2 * de), lambda i: (0, 0)),
            pl.BlockSpec((dg, d + de), lambda i: (0, 0)),
        ],
        out_specs=[
            pl.BlockSpec((bn, d + de), lambda i: (i, 0)),
            pl.BlockSpec((bn, d), lambda i: (i, 0)),
            pl.BlockSpec((bn, de), lambda i: (i, 0)),
            pl.BlockSpec((1, d + de), lambda i: (0, 0)),
        ],
        out_shape=[
            jax.ShapeDtypeStruct((n, d + de), f32),
            jax.ShapeDtypeStruct((n, d), f32),
            jax.ShapeDtypeStruct((n, de), f32),
            jax.ShapeDtypeStruct((1, d + de), f32),
        ],
    )(nodes, globals_, wcat, w4cat)
    gvn = gv[:, :d]
    gve1 = gv[0, d:]

    ch = 40
    ed8 = edges.reshape(e // 8, (8 * de))
    snd2 = senders.astype(jnp.int32).reshape(e // ch, ch)
    rcv2 = receivers.astype(jnp.int32).reshape(e // ch, ch)
    ewb = lax.bitcast_convert_type(edge_weights.astype(f32),
                                   jnp.int32).reshape(e // ch, ch)
    pk = jnp.stack([snd2, rcv2, ewb], axis=1).reshape(3 * e)
    a128, a32, nepk = _make_sc_edge_pass(n, d, e, de)(
        t1, q2, pk, ed8, gve1)

    rpc = ch * de // (8 * de)
    wbd = jnp.kron(jnp.eye(8, dtype=f32), w3e)
    bet = jnp.tile(b_edge, 8).reshape(1, 8 * de)
    ep_blk = 5000
    eblocks = (e // 8) // ep_blk
    ne_packed = pl.pallas_call(
        _make_edge_body(rpc),
        grid=(eblocks,),
        in_specs=[
            pl.BlockSpec((2 * ep_blk, 8 * de), lambda i: (i, 0)),
            pl.BlockSpec((8 * de, 8 * de), lambda i: (0, 0)),
            pl.BlockSpec((1, 8 * de), lambda i: (0, 0)),
        ],
        out_specs=pl.BlockSpec((ep_blk, 8 * de), lambda i: (i, 0)),
        out_shape=jax.ShapeDtypeStruct((e // 8, 8 * de), f32),
    )(nepk, wbd, bet)
    new_edges = ne_packed.reshape(e, de)

    new_nodes, new_global = pl.pallas_call(
        _make_node_body(nblocks),
        grid=(nblocks,),
        in_specs=[
            pl.BlockSpec((bn, d), lambda i: (i, 0)),
            pl.BlockSpec((bn, d), lambda i: (i + nblocks, 0)),
            pl.BlockSpec((bn, 128), lambda i: (i, 0)),
            pl.BlockSpec((bn, 128), lambda i: (i + nblocks, 0)),
            pl.BlockSpec((bn, d), lambda i: (i, 0)),
            pl.BlockSpec((bn, d), lambda i: (i, 0)),
            pl.BlockSpec((de, d), lambda i: (0, 0)),
            pl.BlockSpec((1, d), lambda i: (0, 0)),
            pl.BlockSpec((1, d), lambda i: (0, 0)),
            pl.BlockSpec((1, dg), lambda i: (0, 0)),
            pl.BlockSpec((dg, dg), lambda i: (0, 0)),
            pl.BlockSpec((1, dg), lambda i: (0, 0)),
            pl.BlockSpec((d, dg), lambda i: (0, 0)),
            pl.BlockSpec((1, dg), lambda i: (0, 0)),
            pl.BlockSpec((de, dg), lambda i: (0, 0)),
            pl.BlockSpec((1, dg), lambda i: (0, 0)),
            pl.BlockSpec((3 * dg, dg), lambda i: (0, 0)),
            pl.BlockSpec((1, dg), lambda i: (0, 0)),
        ],
        out_specs=[
            pl.BlockSpec((bn, d), lambda i: (i, 0)),
            pl.BlockSpec((1, dg), lambda i: (0, 0)),
        ],
        out_shape=[
            jax.ShapeDtypeStruct((n, d), f32),
            jax.ShapeDtypeStruct((1, dg), f32),
        ],
        scratch_shapes=[
            pltpu.VMEM((1, d), f32),
            pltpu.VMEM((1, de), f32),
        ],
    )(a128, a128, a32, a32, p2, nodes, w3n, gvn, b_node.reshape(1, d),
      globals_, W_g, b_g.reshape(1, dg), W_gn, b_gn.reshape(1, dg),
      W_ge, b_ge.reshape(1, dg), W_fg, b_fg.reshape(1, dg))

    return (new_nodes, new_edges, new_global)


# speedup vs baseline: 4.3164x; 1.1414x over previous
import functools

import jax
import jax.numpy as jnp
from jax import lax
from jax.experimental import pallas as pl
from jax.experimental.pallas import tpu as pltpu
from jax.experimental.pallas import tpu_sc as plsc

_NC = 2
_NS = 16
_PW = 160


def _pre_body(x_ref, g_ref, wcat_ref, w4_ref, t1_ref, p2_ref, q2_ref, gv_ref):
    y = jnp.dot(x_ref[...], wcat_ref[...], preferred_element_type=jnp.float32)
    d = p2_ref.shape[1]
    de = q2_ref.shape[1]
    t1_ref[...] = y[:, :d + de]
    p2_ref[...] = y[:, d + de:2 * d + de]
    q2_ref[...] = y[:, 2 * d + de:]

    @pl.when(pl.program_id(0) == 0)
    def _():
        gv_ref[...] = jnp.dot(g_ref[...], w4_ref[...],
                              preferred_element_type=jnp.float32)


def _make_sc_edge_pass(n, d, e, de):
    nw = _NC * _NS
    epw = e // nw
    ch = 40
    nit = epw // ch
    nch = n // ch
    npt = -(-nch // _NS)
    assert nit % 2 == 0

    mesh = plsc.VectorSubcoreMesh(core_axis_name="c", subcore_axis_name="s",
                                  num_cores=_NC, num_subcores=_NS)

    rpc = ch * de // 128

    @functools.partial(
        pl.kernel,
    out_type=(
            jax.ShapeDtypeStruct((_NC * n, d), jnp.float32),
            jax.ShapeDtypeStruct((_NC * n, 128), jnp.float32),
            jax.ShapeDtypeStruct((2 * (e // 8), 128), jnp.float32),
        ),
        mesh=mesh,
        compiler_params=pltpu.CompilerParams(use_tc_tiling_on_sc=False),
        scratch_types=[
            pltpu.VMEM_SHARED((n, _PW), jnp.float32),
            pltpu.VMEM((3 * ch,), jnp.int32),
            pltpu.VMEM((3 * ch,), jnp.int32),
            pltpu.VMEM((ch,), jnp.int32),
            pltpu.VMEM((ch,), jnp.int32),
            pltpu.VMEM((ch, d + de), jnp.float32),
            pltpu.VMEM((ch, d + de), jnp.float32),
            pltpu.VMEM((ch, de), jnp.float32),
            pltpu.VMEM((ch, de), jnp.float32),
            pltpu.VMEM((rpc, 128), jnp.float32),
            pltpu.VMEM((rpc, 128), jnp.float32),
            pltpu.VMEM((ch, _PW), jnp.float32),
            pltpu.VMEM((ch, _PW), jnp.float32),
            pltpu.VMEM((2 * rpc, 128), jnp.float32),
            pltpu.VMEM((de,), jnp.float32),
            pltpu.SemaphoreType.DMA,
            pltpu.SemaphoreType.DMA,
            pltpu.SemaphoreType.DMA,
            pltpu.SemaphoreType.DMA,
        ],
    )
    def sc_edge_pass(t1_hbm, q2_hbm, pk_hbm, ed8_hbm, gve_hbm,
                     a128_out, a32_out, ne_out,
                     acc, pb0, pb1, rb0, rb1, tb0, tb1, qb0, qb1,
                     eb0, eb1, yv0, yv1, nv, gvb, sem0, sem1, ssem0, ssem1):
        cid = lax.axis_index("c")
        sid = lax.axis_index("s")
        wid = cid * _NS + sid
        row0 = wid * nit

        @pl.loop(0, ch)
        def _(r):
            for c in range(_PW // 16):
                yv0[r, pl.ds(c * 16, 16)] = jnp.zeros((16,), jnp.float32)

        @pl.loop(0, npt)
        def _(z):
            k = sid + z * _NS

            @pl.when(k < nch)
            def _():
                pltpu.sync_copy(yv0, acc.at[pl.ds(k * ch, ch)])

        plsc.subcore_barrier()

        lane = lax.iota(jnp.int32, 16)
        onehot1 = jnp.where(lane == 1, 1.0, 0.0).astype(jnp.float32)
        pltpu.sync_copy(gve_hbm, gvb)
        gvec = gvb[...]

        def load_issue(row, pb, tb, qb, eb, sm):
            pltpu.sync_copy(pk_hbm.at[pl.ds(row * 3 * ch, 3 * ch)], pb)
            pltpu.async_copy(t1_hbm.at[pb.at[pl.ds(0, ch)]], tb, sm)
            pltpu.async_copy(q2_hbm.at[pb.at[pl.ds(ch, ch)]], qb, sm)
            pltpu.async_copy(ed8_hbm.at[pl.ds(row * rpc, rpc)], eb, sm)

        def drain(row, pb, tb, qb, eb, sm):
            pltpu.make_async_copy(t1_hbm.at[pb.at[pl.ds(0, ch)]], tb,
                                  sm).wait()
            pltpu.make_async_copy(q2_hbm.at[pb.at[pl.ds(ch, ch)]], qb,
                                  sm).wait()
            pltpu.make_async_copy(ed8_hbm.at[pl.ds(row * rpc, rpc)], eb,
                                  sm).wait()

        def combine(row, pb, tb, qb, eb, yv, rbuf, ssem):
            @pl.when(row >= row0 + 2)
            def _():
                pltpu.make_async_copy(yv, acc.at[rbuf], ssem).wait()

            rbuf[pl.ds(0, 16)] = pb[pl.ds(ch, 16)]
            rbuf[pl.ds(16, 16)] = pb[pl.ds(ch + 16, 16)]
            rbuf[pl.ds(24, 16)] = pb[pl.ds(ch + 24, 16)]
            for off, lo in ((0, 0), (16, 0), (24, 8)):
                wv16 = lax.bitcast_convert_type(pb[pl.ds(2 * ch + off, 16)],
                                                jnp.float32)
                for l in range(lo, 16):
                    i = off + l
                    w = wv16[l]
                    wv = jnp.full((16,), w, jnp.float32)
                    for c in range(d // 16):
                        yv[i, pl.ds(c * 16, 16)] = (
                            tb[i, pl.ds(c * 16, 16)] * wv)
                    ewe = eb[i // 8, pl.ds((i % 8) * 16, 16)] * wv
                    yv[i, pl.ds(d, 16)] = ewe
                    nv[rpc + i // 8, pl.ds((i % 8) * 16, 16)] = ewe
                    yv[i, pl.ds(d + 16, 16)] = jnp.where(lane == 0, w,
                                                         onehot1)
                    nv[i // 8, pl.ds((i % 8) * 16, 16)] = (
                        (tb[i, pl.ds(d, 16)] + qb[i, :] + gvec) * wv)
            pltpu.async_copy(yv, acc.at[rbuf], ssem, add=True)
            pltpu.sync_copy(nv, ne_out.at[pl.ds(row * 2 * rpc, 2 * rpc)])

        load_issue(row0, pb0, tb0, qb0, eb0, sem0)

        @pl.loop(0, nit, step=2)
        def _(j):
            ra = row0 + j
            rb_ = ra + 1
            load_issue(rb_, pb1, tb1, qb1, eb1, sem1)
            drain(ra, pb0, tb0, qb0, eb0, sem0)
            combine(ra, pb0, tb0, qb0, eb0, yv0, rb0, ssem0)
            rn = jnp.where(j + 2 >= nit, row0, ra + 2)
            load_issue(rn, pb0, tb0, qb0, eb0, sem0)
            drain(rb_, pb1, tb1, qb1, eb1, sem1)
            combine(rb_, pb1, tb1, qb1, eb1, yv1, rb1, ssem1)

        drain(row0, pb0, tb0, qb0, eb0, sem0)
        pltpu.make_async_copy(yv0, acc.at[rb0], ssem0).wait()
        pltpu.make_async_copy(yv1, acc.at[rb1], ssem1).wait()
        plsc.subcore_barrier()

        rpt = n // _NS
        r0 = sid * rpt
        pltpu.sync_copy(acc.at[pl.ds(r0, rpt), pl.ds(0, d)],
                        a128_out.at[pl.ds(cid * n + r0, rpt)])
        pltpu.sync_copy(acc.at[pl.ds(r0, rpt), pl.ds(d, 2 * de)],
                        a32_out.at[pl.ds(cid * n + r0, rpt),
                                   pl.ds(0, 2 * de)])

    return sc_edge_pass


def _make_edge_body(rpc):
    def _edge_body(ne_ref, wbd_ref, bet_ref, out_ref):
        x = ne_ref[...]
        g = x.reshape(-1, 2, rpc, x.shape[-1])
        ne1 = g[:, 0].reshape(-1, x.shape[-1])
        ne2 = g[:, 1].reshape(-1, x.shape[-1])
        r = jnp.dot(ne2, wbd_ref[...], preferred_element_type=jnp.float32)
        out_ref[...] = ne1 + r + bet_ref[...]

    return _edge_body


def _make_node_body(nblocks):
    def _node_body(acca_ref, accb_ref, ta_ref, tb_ref, p2_ref, x_ref,
                   w3n_ref, gvn_ref,
                   bn_ref, glob_ref, wg_ref, bg_ref, wgn_ref, bgn_ref,
                   wge_ref, bge_ref, wfg_ref, bfg_ref,
                   nn_ref, ng_ref, nsum_ref, esum_ref):
        i = pl.program_id(0)
        a128 = acca_ref[...] + accb_ref[...]
        t = ta_ref[...] + tb_ref[...]
        d = p2_ref.shape[1]
        de = esum_ref.shape[1]
        e16 = t[:, :de]
        s = t[:, de:de + 1]
        deg = t[:, de + 1:de + 2]
        nn_ref[...] = (a128
                       + jnp.dot(e16, w3n_ref[...],
                                 preferred_element_type=jnp.float32)
                       + s * (p2_ref[...] + gvn_ref[...])
                       + deg * bn_ref[...])
        bn_sum = jnp.sum(x_ref[...], axis=0, keepdims=True)
        be_sum = jnp.sum(e16, axis=0, keepdims=True)

        @pl.when(i == 0)
        def _():
            nsum_ref[...] = bn_sum
            esum_ref[...] = be_sum

        @pl.when(i > 0)
        def _():
            nsum_ref[...] += bn_sum
            esum_ref[...] += be_sum

        @pl.when(i == nblocks - 1)
        def _():
            tg = jnp.dot(glob_ref[...], wg_ref[...],
                         preferred_element_type=jnp.float32) + bg_ref[...]
            tn = jnp.dot(nsum_ref[...], wgn_ref[...],
                         preferred_element_type=jnp.float32) + bgn_ref[...]
            te = jnp.dot(esum_ref[...], wge_ref[...],
                         preferred_element_type=jnp.float32) + bge_ref[...]
            fin = jnp.concatenate([tg, tn, te], axis=1)
            ng_ref[...] = jnp.dot(fin, wfg_ref[...],
                                  preferred_element_type=jnp.float32) + bfg_ref[...]

    return _node_body


def kernel(nodes, edges, globals_, senders, receivers, n_node, n_edge,
           edge_weights, W_node, b_node, W_edge, b_edge, W_gn, b_gn,
           W_ge, b_ge, W_g, b_g, W_fg, b_fg):
    n, d = nodes.shape
    e, de = edges.shape
    dg = globals_.shape[1]
    f32 = jnp.float32

    wcat = jnp.concatenate(
        [W_node[:d], W_edge[:d], W_node[d:2 * d], W_edge[d:2 * d]], axis=1)
    w4cat = jnp.concatenate(
        [W_node[2 * d + de:], W_edge[2 * d + de:]], axis=1)
    w3n = W_node[2 * d:2 * d + de]
    w3e = W_edge[2 * d:2 * d + de]

    bn = 2000
    nblocks = n // bn
    t1, p2, q2, gv = pl.pallas_call(
        _pre_body,
        grid=(nblocks,),
        in_specs=[
            pl.BlockSpec((bn, d), lambda i: (i, 0)),
            pl.BlockSpec((1, dg), lambda i: (0, 0)),
            pl.BlockSpec((d, 2 * d + 2 * de), lambda i: (0, 0)),
            pl.BlockSpec((dg, d + de), lambda i: (0, 0)),
        ],
        out_specs=[
            pl.BlockSpec((bn, d + de), lambda i: (i, 0)),
            pl.BlockSpec((bn, d), lambda i: (i, 0)),
            pl.BlockSpec((bn, de), lambda i: (i, 0)),
            pl.BlockSpec((1, d + de), lambda i: (0, 0)),
        ],
        out_shape=[
            jax.ShapeDtypeStruct((n, d + de), f32),
            jax.ShapeDtypeStruct((n, d), f32),
            jax.ShapeDtypeStruct((n, de), f32),
            jax.ShapeDtypeStruct((1, d + de), f32),
        ],
    )(nodes, globals_, wcat, w4cat)
    gvn = gv[:, :d]
    gve1 = gv[0, d:]

    ch = 40
    ed8 = edges.reshape(e // 8, (8 * de))
    snd2 = senders.astype(jnp.int32).reshape(e // ch, ch)
    rcv2 = receivers.astype(jnp.int32).reshape(e // ch, ch)
    ewb = lax.bitcast_convert_type(edge_weights.astype(f32),
                                   jnp.int32).reshape(e // ch, ch)
    pk = jnp.concatenate([snd2, rcv2, ewb], axis=1).reshape(3 * e)
    a128, a32, nepk = _make_sc_edge_pass(n, d, e, de)(
        t1, q2, pk, ed8, gve1)

    rpc = ch * de // (8 * de)
    wbd = jnp.kron(jnp.eye(8, dtype=f32), w3e)
    bet = jnp.tile(b_edge, 8).reshape(1, 8 * de)
    ep_blk = 5000
    eblocks = (e // 8) // ep_blk
    ne_packed = pl.pallas_call(
        _make_edge_body(rpc),
        grid=(eblocks,),
        in_specs=[
            pl.BlockSpec((2 * ep_blk, 8 * de), lambda i: (i, 0)),
            pl.BlockSpec((8 * de, 8 * de), lambda i: (0, 0)),
            pl.BlockSpec((1, 8 * de), lambda i: (0, 0)),
        ],
        out_specs=pl.BlockSpec((ep_blk, 8 * de), lambda i: (i, 0)),
        out_shape=jax.ShapeDtypeStruct((e // 8, 8 * de), f32),
    )(nepk, wbd, bet)
    new_edges = ne_packed.reshape(e, de)

    new_nodes, new_global = pl.pallas_call(
        _make_node_body(nblocks),
        grid=(nblocks,),
        in_specs=[
            pl.BlockSpec((bn, d), lambda i: (i, 0)),
            pl.BlockSpec((bn, d), lambda i: (i + nblocks, 0)),
            pl.BlockSpec((bn, 128), lambda i: (i, 0)),
            pl.BlockSpec((bn, 128), lambda i: (i + nblocks, 0)),
            pl.BlockSpec((bn, d), lambda i: (i, 0)),
            pl.BlockSpec((bn, d), lambda i: (i, 0)),
            pl.BlockSpec((de, d), lambda i: (0, 0)),
            pl.BlockSpec((1, d), lambda i: (0, 0)),
            pl.BlockSpec((1, d), lambda i: (0, 0)),
            pl.BlockSpec((1, dg), lambda i: (0, 0)),
            pl.BlockSpec((dg, dg), lambda i: (0, 0)),
            pl.BlockSpec((1, dg), lambda i: (0, 0)),
            pl.BlockSpec((d, dg), lambda i: (0, 0)),
            pl.BlockSpec((1, dg), lambda i: (0, 0)),
            pl.BlockSpec((de, dg), lambda i: (0, 0)),
            pl.BlockSpec((1, dg), lambda i: (0, 0)),
            pl.BlockSpec((3 * dg, dg), lambda i: (0, 0)),
            pl.BlockSpec((1, dg), lambda i: (0, 0)),
        ],
        out_specs=[
            pl.BlockSpec((bn, d), lambda i: (i, 0)),
            pl.BlockSpec((1, dg), lambda i: (0, 0)),
        ],
        out_shape=[
            jax.ShapeDtypeStruct((n, d), f32),
            jax.ShapeDtypeStruct((1, dg), f32),
        ],
        scratch_shapes=[
            pltpu.VMEM((1, d), f32),
            pltpu.VMEM((1, de), f32),
        ],
    )(a128, a128, a32, a32, p2, nodes, w3n, gvn, b_node.reshape(1, d),
      globals_, W_g, b_g.reshape(1, dg), W_gn, b_gn.reshape(1, dg),
      W_ge, b_ge.reshape(1, dg), W_fg, b_fg.reshape(1, dg))

    return (new_nodes, new_edges, new_global)


# speedup vs baseline: 4.4012x; 1.0196x over previous
import functools

import jax
import jax.numpy as jnp
from jax import lax
from jax.experimental import pallas as pl
from jax.experimental.pallas import tpu as pltpu
from jax.experimental.pallas import tpu_sc as plsc

_NC = 2
_NS = 16
_PW = 160


def _pre_body(x_ref, g_ref, wcat_ref, w4_ref, t1_ref, p2_ref, q2_ref, gv_ref):
    y = jnp.dot(x_ref[...], wcat_ref[...], preferred_element_type=jnp.float32)
    d = p2_ref.shape[1]
    de = q2_ref.shape[1]
    t1_ref[...] = y[:, :d + de]
    p2_ref[...] = y[:, d + de:2 * d + de]
    q2_ref[...] = y[:, 2 * d + de:]

    @pl.when(pl.program_id(0) == 0)
    def _():
        gv_ref[...] = jnp.dot(g_ref[...], w4_ref[...],
                              preferred_element_type=jnp.float32)


def _make_sc_edge_pass(n, d, e, de):
    nw = _NC * _NS
    epw = e // nw
    ch = 40
    nit = epw // ch
    nch = n // ch
    npt = -(-nch // _NS)
    assert nit % 2 == 0

    mesh = plsc.VectorSubcoreMesh(core_axis_name="c", subcore_axis_name="s",
                                  num_cores=_NC, num_subcores=_NS)

    rpc = ch * de // 128

    @functools.partial(
        pl.kernel,
    out_type=(
            jax.ShapeDtypeStruct((_NC * n, d), jnp.float32),
            jax.ShapeDtypeStruct((_NC * n, 128), jnp.float32),
            jax.ShapeDtypeStruct((2 * (e // 8), 128), jnp.float32),
        ),
        mesh=mesh,
        compiler_params=pltpu.CompilerParams(use_tc_tiling_on_sc=False),
        scratch_types=[
            pltpu.VMEM_SHARED((n, _PW), jnp.float32),
            pltpu.VMEM((3 * ch,), jnp.int32),
            pltpu.VMEM((3 * ch,), jnp.int32),
            pltpu.VMEM((ch,), jnp.int32),
            pltpu.VMEM((ch,), jnp.int32),
            pltpu.VMEM((ch, d + de), jnp.float32),
            pltpu.VMEM((ch, d + de), jnp.float32),
            pltpu.VMEM((ch, de), jnp.float32),
            pltpu.VMEM((ch, de), jnp.float32),
            pltpu.VMEM((rpc, 128), jnp.float32),
            pltpu.VMEM((rpc, 128), jnp.float32),
            pltpu.VMEM((ch, _PW), jnp.float32),
            pltpu.VMEM((ch, _PW), jnp.float32),
            pltpu.VMEM((2 * rpc, 128), jnp.float32),
            pltpu.VMEM((2 * rpc, 128), jnp.float32),
            pltpu.VMEM((de,), jnp.float32),
            pltpu.SemaphoreType.DMA,
            pltpu.SemaphoreType.DMA,
            pltpu.SemaphoreType.DMA,
            pltpu.SemaphoreType.DMA,
            pltpu.SemaphoreType.DMA,
            pltpu.SemaphoreType.DMA,
        ],
    )
    def sc_edge_pass(t1_hbm, q2_hbm, pk_hbm, ed8_hbm, gve_hbm,
                     a128_out, a32_out, ne_out,
                     acc, pb0, pb1, rb0, rb1, tb0, tb1, qb0, qb1,
                     eb0, eb1, yv0, yv1, nv0, nv1, gvb, sem0, sem1,
                     ssem0, ssem1, nsem0, nsem1):
        cid = lax.axis_index("c")
        sid = lax.axis_index("s")
        wid = cid * _NS + sid
        row0 = wid * nit

        @pl.loop(0, ch)
        def _(r):
            for c in range(_PW // 16):
                yv0[r, pl.ds(c * 16, 16)] = jnp.zeros((16,), jnp.float32)

        @pl.loop(0, npt)
        def _(z):
            k = sid + z * _NS

            @pl.when(k < nch)
            def _():
                pltpu.sync_copy(yv0, acc.at[pl.ds(k * ch, ch)])

        plsc.subcore_barrier()

        lane = lax.iota(jnp.int32, 16)
        onehot1 = jnp.where(lane == 1, 1.0, 0.0).astype(jnp.float32)
        pltpu.sync_copy(gve_hbm, gvb)
        gvec = gvb[...]

        def load_issue(row, pb, tb, qb, eb, sm):
            pltpu.sync_copy(pk_hbm.at[pl.ds(row * 3 * ch, 3 * ch)], pb)
            pltpu.async_copy(t1_hbm.at[pb.at[pl.ds(0, ch)]], tb, sm)
            pltpu.async_copy(q2_hbm.at[pb.at[pl.ds(ch, ch)]], qb, sm)
            pltpu.async_copy(ed8_hbm.at[pl.ds(row * rpc, rpc)], eb, sm)

        def drain(row, pb, tb, qb, eb, sm):
            pltpu.make_async_copy(t1_hbm.at[pb.at[pl.ds(0, ch)]], tb,
                                  sm).wait()
            pltpu.make_async_copy(q2_hbm.at[pb.at[pl.ds(ch, ch)]], qb,
                                  sm).wait()
            pltpu.make_async_copy(ed8_hbm.at[pl.ds(row * rpc, rpc)], eb,
                                  sm).wait()

        def combine(row, pb, tb, qb, eb, yv, rbuf, nv, ssem, nsem):
            @pl.when(row >= row0 + 2)
            def _():
                pltpu.make_async_copy(yv, acc.at[rbuf], ssem).wait()
                pltpu.make_async_copy(
                    nv, ne_out.at[pl.ds((row - 2) * 2 * rpc, 2 * rpc)],
                    nsem).wait()

            rbuf[pl.ds(0, 16)] = pb[pl.ds(ch, 16)]
            rbuf[pl.ds(16, 16)] = pb[pl.ds(ch + 16, 16)]
            rbuf[pl.ds(24, 16)] = pb[pl.ds(ch + 24, 16)]
            for off, lo in ((0, 0), (16, 0), (24, 8)):
                wv16 = lax.bitcast_convert_type(pb[pl.ds(2 * ch + off, 16)],
                                                jnp.float32)
                for l in range(lo, 16):
                    i = off + l
                    w = wv16[l]
                    wv = jnp.full((16,), w, jnp.float32)
                    for c in range(d // 16):
                        yv[i, pl.ds(c * 16, 16)] = (
                            tb[i, pl.ds(c * 16, 16)] * wv)
                    ewe = eb[i // 8, pl.ds((i % 8) * 16, 16)] * wv
                    yv[i, pl.ds(d, 16)] = ewe
                    nv[rpc + i // 8, pl.ds((i % 8) * 16, 16)] = ewe
                    yv[i, pl.ds(d + 16, 16)] = jnp.where(lane == 0, w,
                                                         onehot1)
                    nv[i // 8, pl.ds((i % 8) * 16, 16)] = (
                        (tb[i, pl.ds(d, 16)] + qb[i, :] + gvec) * wv)
            pltpu.async_copy(yv, acc.at[rbuf], ssem, add=True)
            pltpu.async_copy(nv, ne_out.at[pl.ds(row * 2 * rpc, 2 * rpc)],
                             nsem)

        load_issue(row0, pb0, tb0, qb0, eb0, sem0)

        @pl.loop(0, nit, step=2)
        def _(j):
            ra = row0 + j
            rb_ = ra + 1
            load_issue(rb_, pb1, tb1, qb1, eb1, sem1)
            drain(ra, pb0, tb0, qb0, eb0, sem0)
            combine(ra, pb0, tb0, qb0, eb0, yv0, rb0, nv0, ssem0, nsem0)
            rn = jnp.where(j + 2 >= nit, row0, ra + 2)
            load_issue(rn, pb0, tb0, qb0, eb0, sem0)
            drain(rb_, pb1, tb1, qb1, eb1, sem1)
            combine(rb_, pb1, tb1, qb1, eb1, yv1, rb1, nv1, ssem1, nsem1)

        drain(row0, pb0, tb0, qb0, eb0, sem0)
        pltpu.make_async_copy(yv0, acc.at[rb0], ssem0).wait()
        pltpu.make_async_copy(yv1, acc.at[rb1], ssem1).wait()
        last0 = row0 + nit - 2
        pltpu.make_async_copy(
            nv0, ne_out.at[pl.ds(last0 * 2 * rpc, 2 * rpc)], nsem0).wait()
        pltpu.make_async_copy(
            nv1, ne_out.at[pl.ds((last0 + 1) * 2 * rpc, 2 * rpc)],
            nsem1).wait()
        plsc.subcore_barrier()

        rpt = n // _NS
        r0 = sid * rpt
        pltpu.sync_copy(acc.at[pl.ds(r0, rpt), pl.ds(0, d)],
                        a128_out.at[pl.ds(cid * n + r0, rpt)])
        pltpu.sync_copy(acc.at[pl.ds(r0, rpt), pl.ds(d, 2 * de)],
                        a32_out.at[pl.ds(cid * n + r0, rpt),
                                   pl.ds(0, 2 * de)])

    return sc_edge_pass


def _make_edge_body(rpc):
    def _edge_body(ne_ref, wbd_ref, bet_ref, out_ref):
        x = ne_ref[...]
        g = x.reshape(-1, 2, rpc, x.shape[-1])
        ne1 = g[:, 0].reshape(-1, x.shape[-1])
        ne2 = g[:, 1].reshape(-1, x.shape[-1])
        r = jnp.dot(ne2, wbd_ref[...], preferred_element_type=jnp.float32)
        out_ref[...] = ne1 + r + bet_ref[...]

    return _edge_body


def _make_node_body(nblocks):
    def _node_body(acca_ref, accb_ref, ta_ref, tb_ref, p2_ref, x_ref,
                   w3n_ref, gvn_ref,
                   bn_ref, glob_ref, wg_ref, bg_ref, wgn_ref, bgn_ref,
                   wge_ref, bge_ref, wfg_ref, bfg_ref,
                   nn_ref, ng_ref, nsum_ref, esum_ref):
        i = pl.program_id(0)
        a128 = acca_ref[...] + accb_ref[...]
        t = ta_ref[...] + tb_ref[...]
        d = p2_ref.shape[1]
        de = esum_ref.shape[1]
        e16 = t[:, :de]
        s = t[:, de:de + 1]
        deg = t[:, de + 1:de + 2]
        nn_ref[...] = (a128
                       + jnp.dot(e16, w3n_ref[...],
                                 preferred_element_type=jnp.float32)
                       + s * (p2_ref[...] + gvn_ref[...])
                       + deg * bn_ref[...])
        bn_sum = jnp.sum(x_ref[...], axis=0, keepdims=True)
        be_sum = jnp.sum(e16, axis=0, keepdims=True)

        @pl.when(i == 0)
        def _():
            nsum_ref[...] = bn_sum
            esum_ref[...] = be_sum

        @pl.when(i > 0)
        def _():
            nsum_ref[...] += bn_sum
            esum_ref[...] += be_sum

        @pl.when(i == nblocks - 1)
        def _():
            tg = jnp.dot(glob_ref[...], wg_ref[...],
                         preferred_element_type=jnp.float32) + bg_ref[...]
            tn = jnp.dot(nsum_ref[...], wgn_ref[...],
                         preferred_element_type=jnp.float32) + bgn_ref[...]
            te = jnp.dot(esum_ref[...], wge_ref[...],
                         preferred_element_type=jnp.float32) + bge_ref[...]
            fin = jnp.concatenate([tg, tn, te], axis=1)
            ng_ref[...] = jnp.dot(fin, wfg_ref[...],
                                  preferred_element_type=jnp.float32) + bfg_ref[...]

    return _node_body


def kernel(nodes, edges, globals_, senders, receivers, n_node, n_edge,
           edge_weights, W_node, b_node, W_edge, b_edge, W_gn, b_gn,
           W_ge, b_ge, W_g, b_g, W_fg, b_fg):
    n, d = nodes.shape
    e, de = edges.shape
    dg = globals_.shape[1]
    f32 = jnp.float32

    wcat = jnp.concatenate(
        [W_node[:d], W_edge[:d], W_node[d:2 * d], W_edge[d:2 * d]], axis=1)
    w4cat = jnp.concatenate(
        [W_node[2 * d + de:], W_edge[2 * d + de:]], axis=1)
    w3n = W_node[2 * d:2 * d + de]
    w3e = W_edge[2 * d:2 * d + de]

    bn = 2000
    nblocks = n // bn
    t1, p2, q2, gv = pl.pallas_call(
        _pre_body,
        grid=(nblocks,),
        in_specs=[
            pl.BlockSpec((bn, d), lambda i: (i, 0)),
            pl.BlockSpec((1, dg), lambda i: (0, 0)),
            pl.BlockSpec((d, 2 * d + 2 * de), lambda i: (0, 0)),
            pl.BlockSpec((dg, d + de), lambda i: (0, 0)),
        ],
        out_specs=[
            pl.BlockSpec((bn, d + de), lambda i: (i, 0)),
            pl.BlockSpec((bn, d), lambda i: (i, 0)),
            pl.BlockSpec((bn, de), lambda i: (i, 0)),
            pl.BlockSpec((1, d + de), lambda i: (0, 0)),
        ],
        out_shape=[
            jax.ShapeDtypeStruct((n, d + de), f32),
            jax.ShapeDtypeStruct((n, d), f32),
            jax.ShapeDtypeStruct((n, de), f32),
            jax.ShapeDtypeStruct((1, d + de), f32),
        ],
    )(nodes, globals_, wcat, w4cat)
    gvn = gv[:, :d]
    gve1 = gv[0, d:]

    ch = 40
    ed8 = edges.reshape(e // 8, (8 * de))
    snd2 = senders.astype(jnp.int32).reshape(e // ch, ch)
    rcv2 = receivers.astype(jnp.int32).reshape(e // ch, ch)
    ewb = lax.bitcast_convert_type(edge_weights.astype(f32),
                                   jnp.int32).reshape(e // ch, ch)
    pk = jnp.concatenate([snd2, rcv2, ewb], axis=1).reshape(3 * e)
    a128, a32, nepk = _make_sc_edge_pass(n, d, e, de)(
        t1, q2, pk, ed8, gve1)

    rpc = ch * de // (8 * de)
    wbd = jnp.kron(jnp.eye(8, dtype=f32), w3e)
    bet = jnp.tile(b_edge, 8).reshape(1, 8 * de)
    ep_blk = 5000
    eblocks = (e // 8) // ep_blk
    ne_packed = pl.pallas_call(
        _make_edge_body(rpc),
        grid=(eblocks,),
        in_specs=[
            pl.BlockSpec((2 * ep_blk, 8 * de), lambda i: (i, 0)),
            pl.BlockSpec((8 * de, 8 * de), lambda i: (0, 0)),
            pl.BlockSpec((1, 8 * de), lambda i: (0, 0)),
        ],
        out_specs=pl.BlockSpec((ep_blk, 8 * de), lambda i: (i, 0)),
        out_shape=jax.ShapeDtypeStruct((e // 8, 8 * de), f32),
    )(nepk, wbd, bet)
    new_edges = ne_packed.reshape(e, de)

    new_nodes, new_global = pl.pallas_call(
        _make_node_body(nblocks),
        grid=(nblocks,),
        in_specs=[
            pl.BlockSpec((bn, d), lambda i: (i, 0)),
            pl.BlockSpec((bn, d), lambda i: (i + nblocks, 0)),
            pl.BlockSpec((bn, 128), lambda i: (i, 0)),
            pl.BlockSpec((bn, 128), lambda i: (i + nblocks, 0)),
            pl.BlockSpec((bn, d), lambda i: (i, 0)),
            pl.BlockSpec((bn, d), lambda i: (i, 0)),
            pl.BlockSpec((de, d), lambda i: (0, 0)),
            pl.BlockSpec((1, d), lambda i: (0, 0)),
            pl.BlockSpec((1, d), lambda i: (0, 0)),
            pl.BlockSpec((1, dg), lambda i: (0, 0)),
            pl.BlockSpec((dg, dg), lambda i: (0, 0)),
            pl.BlockSpec((1, dg), lambda i: (0, 0)),
            pl.BlockSpec((d, dg), lambda i: (0, 0)),
            pl.BlockSpec((1, dg), lambda i: (0, 0)),
            pl.BlockSpec((de, dg), lambda i: (0, 0)),
            pl.BlockSpec((1, dg), lambda i: (0, 0)),
            pl.BlockSpec((3 * dg, dg), lambda i: (0, 0)),
            pl.BlockSpec((1, dg), lambda i: (0, 0)),
        ],
        out_specs=[
            pl.BlockSpec((bn, d), lambda i: (i, 0)),
            pl.BlockSpec((1, dg), lambda i: (0, 0)),
        ],
        out_shape=[
            jax.ShapeDtypeStruct((n, d), f32),
            jax.ShapeDtypeStruct((1, dg), f32),
        ],
        scratch_shapes=[
            pltpu.VMEM((1, d), f32),
            pltpu.VMEM((1, de), f32),
        ],
    )(a128, a128, a32, a32, p2, nodes, w3n, gvn, b_node.reshape(1, d),
      globals_, W_g, b_g.reshape(1, dg), W_gn, b_gn.reshape(1, dg),
      W_ge, b_ge.reshape(1, dg), W_fg, b_fg.reshape(1, dg))

    return (new_nodes, new_edges, new_global)
